# Initial kernel scaffold; baseline (speedup 1.0000x reference)
import jax
import jax.numpy as jnp
from jax.experimental import pallas as pl


def kernel(x):
    raise NotImplementedError("write your pallas kernel here")


# speedup vs baseline: 1179.1237x; 1179.1237x over previous
import functools

import jax
import jax.numpy as jnp
import numpy as np
from jax import lax
from jax.experimental import pallas as pl
from jax.experimental.pallas import tpu as pltpu

_CLIP_LIMIT = 4.0
_N_BINS = 256
_GH, _GW = 8, 8
_BANDWIDTH = 1e-3


def _bspline5(t):
    x = jnp.abs(t)
    x2 = x * x
    x4 = x2 * x2
    w1 = 11.0 / 20.0 - x2 / 2.0 + x4 / 4.0 - x4 * x / 12.0
    w2 = (17.0 / 40.0 + 5.0 * x / 8.0 - 7.0 * x2 / 4.0 + 5.0 * x2 * x / 4.0
          - 3.0 * x4 / 8.0 + x4 * x / 24.0)
    w3 = (3.0 - x) ** 5 / 120.0
    return jnp.where(x < 1.0, w1, jnp.where(x < 2.0, w2,
                     jnp.where(x < 3.0, w3, 0.0)))


def _bspline5_np(t):
    x = np.abs(t)
    x2 = x * x
    x4 = x2 * x2
    w1 = 11.0 / 20.0 - x2 / 2.0 + x4 / 4.0 - x4 * x / 12.0
    w2 = (17.0 / 40.0 + 5.0 * x / 8.0 - 7.0 * x2 / 4.0 + 5.0 * x2 * x / 4.0
          - 3.0 * x4 / 8.0 + x4 * x / 24.0)
    w3 = (3.0 - x) ** 5 / 120.0
    return np.where(x < 1.0, w1, np.where(x < 2.0, w2,
                    np.where(x < 3.0, w3, 0.0)))


@functools.lru_cache(maxsize=None)
def _spatial_weights(n_pix, n_grid):
    c = np.linspace(-0.5 - 0.25 / n_grid, n_grid - 1 + 0.5 + 0.25 / n_grid,
                    n_pix, dtype=np.float64)
    base = np.floor(c).astype(np.int64)
    W = np.zeros((n_pix, n_grid), np.float64)
    for i in range(6):
        n = base - 2 + i
        w = _bspline5_np(c - n)
        m = np.remainder(n, 2 * n_grid)
        refl = np.where(m >= n_grid, 2 * n_grid - 1 - m, m)
        np.add.at(W, (np.arange(n_pix), refl), w)
    return W.astype(np.float32)


def _hist_cdf_body(x_ref, cdf_ref, *, vox, n_bins, limit):
    tile = x_ref[0]
    bins = lax.broadcasted_iota(jnp.int32, (1, 1, n_bins), 2).astype(
        jnp.float32) * (1.0 / (n_bins - 1.0))
    z = (tile[:, :, None] - bins) * (1.0 / _BANDWIDTH)
    w = jnp.exp(-0.5 * (z * z))
    pdf = jnp.sum(w.reshape(vox, n_bins), axis=0, keepdims=True) / vox
    pdf = pdf / (jnp.sum(pdf) + 1e-10)
    histos = jnp.minimum(pdf * vox, limit)
    clipped = vox - jnp.sum(histos)
    residual = jnp.remainder(clipped, float(n_bins))
    redist = (clipped - residual) / n_bins
    bidx = lax.broadcasted_iota(jnp.int32, (1, n_bins), 1).astype(jnp.float32)
    histos = histos + redist + (bidx < residual).astype(jnp.float32)
    ii = lax.broadcasted_iota(jnp.int32, (n_bins, n_bins), 0)
    jj = lax.broadcasted_iota(jnp.int32, (n_bins, n_bins), 1)
    tri = (ii <= jj).astype(jnp.float32)
    cdf = jnp.dot(histos, tri, preferred_element_type=jnp.float32)
    cdf_ref[0] = jnp.clip(cdf * ((n_bins - 1.0) / vox), 0.0, n_bins - 1.0)


def _resample_body(x_ref, vol_ref, roww_ref, colw_ref, out_ref, *,
                   rows, w_pix, n_bins, n_tiles):
    f = x_ref[0] * (n_bins - 1.0)
    nodes = lax.broadcasted_iota(jnp.int32, (1, 1, n_bins + 4), 2).astype(
        jnp.float32) - 2.0
    wv = _bspline5(f[:, :, None] - nodes)
    wv2 = wv.reshape(rows * w_pix, n_bins + 4)
    vol = vol_ref[0]
    c = lax.dot_general(wv2, vol, (((1,), (1,)), ((), ())),
                        preferred_element_type=jnp.float32)
    c3 = c.reshape(rows, w_pix, n_tiles)
    mask = roww_ref[...][:, None, :] * colw_ref[...][None, :, :]
    out_ref[0] = jnp.sum(c3 * mask, axis=2)


def _finalize_body(x_ref, o_ref):
    x = x_ref[...]
    mn = jnp.min(x)
    mx = jnp.max(x)
    o_ref[...] = (x - mn) / (mx - mn + 1e-10)


def kernel(x):
    B, C, H, W = x.shape
    th, tw = H // _GH, W // _GW
    vox = th * tw
    n_tiles = _GH * _GW
    bc = B * C
    nbt = bc * n_tiles
    limit = max(_CLIP_LIMIT * vox // _N_BINS, 1)
    nbe = _N_BINS + 4

    xt = x.reshape(bc, _GH, th, _GW, tw).transpose(0, 1, 3, 2, 4)
    xt = xt.reshape(nbt, 8, vox // 8)
    cdfs = pl.pallas_call(
        functools.partial(_hist_cdf_body, vox=vox, n_bins=_N_BINS,
                          limit=float(limit)),
        grid=(nbt,),
        in_specs=[pl.BlockSpec((1, 8, vox // 8), lambda i: (i, 0, 0))],
        out_specs=pl.BlockSpec((1, 1, _N_BINS), lambda i: (i, 0, 0)),
        out_shape=jax.ShapeDtypeStruct((nbt, 1, _N_BINS), jnp.float32),
        compiler_params=pltpu.CompilerParams(
            dimension_semantics=("parallel",)),
    )(xt)

    vol = cdfs.reshape(bc, n_tiles, _N_BINS)
    vol_e = jnp.concatenate(
        [vol[:, :, 1:2], vol[:, :, 0:1], vol,
         vol[:, :, _N_BINS - 1:_N_BINS], vol[:, :, _N_BINS - 2:_N_BINS - 1]],
        axis=-1)

    roww = np.repeat(_spatial_weights(H, _GH), _GW, axis=1)
    colw = np.tile(_spatial_weights(W, _GW), (1, _GH))

    ROWS = 8
    n_rb = H // ROWS
    out = pl.pallas_call(
        functools.partial(_resample_body, rows=ROWS, w_pix=W,
                          n_bins=_N_BINS, n_tiles=n_tiles),
        grid=(bc, n_rb),
        in_specs=[
            pl.BlockSpec((1, ROWS, W), lambda b, r: (b, r, 0)),
            pl.BlockSpec((1, n_tiles, nbe), lambda b, r: (b, 0, 0)),
            pl.BlockSpec((ROWS, n_tiles), lambda b, r: (r, 0)),
            pl.BlockSpec((W, n_tiles), lambda b, r: (0, 0)),
        ],
        out_specs=pl.BlockSpec((1, ROWS, W), lambda b, r: (b, r, 0)),
        out_shape=jax.ShapeDtypeStruct((bc, H, W), jnp.float32),
        compiler_params=pltpu.CompilerParams(
            dimension_semantics=("parallel", "parallel")),
    )(x.reshape(bc, H, W), vol_e, jnp.asarray(roww), jnp.asarray(colw))

    outn = pl.pallas_call(
        _finalize_body,
        out_shape=jax.ShapeDtypeStruct((bc, H, W), jnp.float32),
    )(out)
    return outn.reshape(B, C, H, W)


# speedup vs baseline: 1376.0911x; 1.1670x over previous
import functools

import jax
import jax.numpy as jnp
import numpy as np
from jax import lax
from jax.experimental import pallas as pl
from jax.experimental.pallas import tpu as pltpu

_CLIP_LIMIT = 4.0
_N_BINS = 256
_GH, _GW = 8, 8
_BANDWIDTH = 1e-3


def _bspline5(t):
    x = jnp.abs(t)
    x2 = x * x
    x4 = x2 * x2
    w1 = 11.0 / 20.0 - x2 / 2.0 + x4 / 4.0 - x4 * x / 12.0
    w2 = (17.0 / 40.0 + 5.0 * x / 8.0 - 7.0 * x2 / 4.0 + 5.0 * x2 * x / 4.0
          - 3.0 * x4 / 8.0 + x4 * x / 24.0)
    w3 = (3.0 - x) ** 5 / 120.0
    return jnp.where(x < 1.0, w1, jnp.where(x < 2.0, w2,
                     jnp.where(x < 3.0, w3, 0.0)))


def _bspline5_np(t):
    x = np.abs(t)
    x2 = x * x
    x4 = x2 * x2
    w1 = 11.0 / 20.0 - x2 / 2.0 + x4 / 4.0 - x4 * x / 12.0
    w2 = (17.0 / 40.0 + 5.0 * x / 8.0 - 7.0 * x2 / 4.0 + 5.0 * x2 * x / 4.0
          - 3.0 * x4 / 8.0 + x4 * x / 24.0)
    w3 = (3.0 - x) ** 5 / 120.0
    return np.where(x < 1.0, w1, np.where(x < 2.0, w2,
                    np.where(x < 3.0, w3, 0.0)))


@functools.lru_cache(maxsize=None)
def _spatial_weights(n_pix, n_grid):
    c = np.linspace(-0.5 - 0.25 / n_grid, n_grid - 1 + 0.5 + 0.25 / n_grid,
                    n_pix, dtype=np.float64)
    base = np.floor(c).astype(np.int64)
    W = np.zeros((n_pix, n_grid), np.float64)
    for i in range(6):
        n = base - 2 + i
        w = _bspline5_np(c - n)
        m = np.remainder(n, 2 * n_grid)
        refl = np.where(m >= n_grid, 2 * n_grid - 1 - m, m)
        np.add.at(W, (np.arange(n_pix), refl), w)
    return W.astype(np.float32)


def _hist_cdf_body(x_ref, cdf_ref, *, vox, n_bins, limit):
    tile = x_ref[0]
    bins = lax.broadcasted_iota(jnp.int32, (1, 1, n_bins), 2).astype(
        jnp.float32) * (1.0 / (n_bins - 1.0))
    z = (tile[:, :, None] - bins) * (1.0 / _BANDWIDTH)
    w = jnp.exp(-0.5 * (z * z))
    pdf = jnp.sum(w.reshape(vox, n_bins), axis=0, keepdims=True) / vox
    pdf = pdf / (jnp.sum(pdf) + 1e-10)
    histos = jnp.minimum(pdf * vox, limit)
    clipped = vox - jnp.sum(histos)
    residual = jnp.remainder(clipped, float(n_bins))
    redist = (clipped - residual) / n_bins
    bidx = lax.broadcasted_iota(jnp.int32, (1, n_bins), 1).astype(jnp.float32)
    histos = histos + redist + (bidx < residual).astype(jnp.float32)
    ii = lax.broadcasted_iota(jnp.int32, (n_bins, n_bins), 0)
    jj = lax.broadcasted_iota(jnp.int32, (n_bins, n_bins), 1)
    tri = (ii <= jj).astype(jnp.float32)
    cdf = jnp.dot(histos, tri, preferred_element_type=jnp.float32)
    cdf_ref[0] = jnp.clip(cdf * ((n_bins - 1.0) / vox), 0.0, n_bins - 1.0)


def _resample_body(x_ref, vol_ref, volc_ref, roww_ref, colw_ref, out_ref, *,
                   rows, w_pix, n_bins, n_tiles):
    f = x_ref[0] * (n_bins - 1.0)
    nodes = lax.broadcasted_iota(jnp.int32, (1, 1, n_bins), 2).astype(
        jnp.float32)
    wv = _bspline5(f[:, :, None] - nodes)
    wv2 = wv.reshape(rows * w_pix, n_bins)
    vol = vol_ref[0]
    c = lax.dot_general(wv2, vol, (((1,), (1,)), ((), ())),
                        preferred_element_type=jnp.float32)
    ce = jnp.stack([_bspline5(f + 1.0), _bspline5(f + 2.0),
                    _bspline5(f - float(n_bins)),
                    _bspline5(f - float(n_bins + 1))],
                   axis=-1)
    ce2 = ce.reshape(rows * w_pix, 4)
    volc = volc_ref[0]
    c = c + lax.dot_general(ce2, volc, (((1,), (1,)), ((), ())),
                            preferred_element_type=jnp.float32)
    c3 = c.reshape(rows, w_pix, n_tiles)
    mask = roww_ref[...][:, None, :] * colw_ref[...][None, :, :]
    out_ref[0] = jnp.sum(c3 * mask, axis=2)


def _finalize_body(x_ref, o_ref):
    x = x_ref[...]
    mn = jnp.min(x)
    mx = jnp.max(x)
    o_ref[...] = (x - mn) / (mx - mn + 1e-10)


def kernel(x):
    B, C, H, W = x.shape
    th, tw = H // _GH, W // _GW
    vox = th * tw
    n_tiles = _GH * _GW
    bc = B * C
    nbt = bc * n_tiles
    limit = max(_CLIP_LIMIT * vox // _N_BINS, 1)

    xt = x.reshape(bc, _GH, th, _GW, tw).transpose(0, 1, 3, 2, 4)
    xt = xt.reshape(nbt, 8, vox // 8)
    cdfs = pl.pallas_call(
        functools.partial(_hist_cdf_body, vox=vox, n_bins=_N_BINS,
                          limit=float(limit)),
        grid=(nbt,),
        in_specs=[pl.BlockSpec((1, 8, vox // 8), lambda i: (i, 0, 0))],
        out_specs=pl.BlockSpec((1, 1, _N_BINS), lambda i: (i, 0, 0)),
        out_shape=jax.ShapeDtypeStruct((nbt, 1, _N_BINS), jnp.float32),
        compiler_params=pltpu.CompilerParams(
            dimension_semantics=("parallel",)),
    )(xt)

    vol = cdfs.reshape(bc, n_tiles, _N_BINS)
    vol_c = jnp.concatenate(
        [vol[:, :, 0:1], vol[:, :, 1:2],
         vol[:, :, _N_BINS - 1:_N_BINS], vol[:, :, _N_BINS - 2:_N_BINS - 1]],
        axis=-1)

    roww = np.repeat(_spatial_weights(H, _GH), _GW, axis=1)
    colw = np.tile(_spatial_weights(W, _GW), (1, _GH))

    ROWS = 8
    n_rb = H // ROWS
    out = pl.pallas_call(
        functools.partial(_resample_body, rows=ROWS, w_pix=W,
                          n_bins=_N_BINS, n_tiles=n_tiles),
        grid=(bc, n_rb),
        in_specs=[
            pl.BlockSpec((1, ROWS, W), lambda b, r: (b, r, 0)),
            pl.BlockSpec((1, n_tiles, _N_BINS), lambda b, r: (b, 0, 0)),
            pl.BlockSpec((1, n_tiles, 4), lambda b, r: (b, 0, 0)),
            pl.BlockSpec((ROWS, n_tiles), lambda b, r: (r, 0)),
            pl.BlockSpec((W, n_tiles), lambda b, r: (0, 0)),
        ],
        out_specs=pl.BlockSpec((1, ROWS, W), lambda b, r: (b, r, 0)),
        out_shape=jax.ShapeDtypeStruct((bc, H, W), jnp.float32),
        compiler_params=pltpu.CompilerParams(
            dimension_semantics=("parallel", "parallel")),
    )(x.reshape(bc, H, W), vol, vol_c, jnp.asarray(roww), jnp.asarray(colw))

    outn = pl.pallas_call(
        _finalize_body,
        out_shape=jax.ShapeDtypeStruct((bc, H, W), jnp.float32),
    )(out)
    return outn.reshape(B, C, H, W)


# speedup vs baseline: 1569.2117x; 1.1403x over previous
import functools

import jax
import jax.numpy as jnp
import numpy as np
from jax import lax
from jax.experimental import pallas as pl
from jax.experimental.pallas import tpu as pltpu

_CLIP_LIMIT = 4.0
_N_BINS = 256
_GH, _GW = 8, 8
_BANDWIDTH = 1e-3


def _bspline5(t):
    x = jnp.abs(t)
    x2 = x * x
    x4 = x2 * x2
    w1 = 11.0 / 20.0 - x2 / 2.0 + x4 / 4.0 - x4 * x / 12.0
    w2 = (17.0 / 40.0 + 5.0 * x / 8.0 - 7.0 * x2 / 4.0 + 5.0 * x2 * x / 4.0
          - 3.0 * x4 / 8.0 + x4 * x / 24.0)
    w3 = (3.0 - x) ** 5 / 120.0
    return jnp.where(x < 1.0, w1, jnp.where(x < 2.0, w2,
                     jnp.where(x < 3.0, w3, 0.0)))


def _bspline5_horner(t):
    x = jnp.abs(t)
    lt1 = x < 1.0
    lt2 = x < 2.0

    def pick(a, b, c):
        return jnp.where(lt1, a, jnp.where(lt2, b, c))

    c0 = pick(0.55, 0.425, 2.025)
    c1 = pick(0.0, 0.625, -3.375)
    c2 = pick(-0.5, -1.75, 2.25)
    c3 = pick(0.0, 1.25, -0.75)
    c4 = pick(0.25, -0.375, 0.125)
    c5 = pick(-1.0 / 12.0, 1.0 / 24.0, -1.0 / 120.0)
    p = ((((c5 * x + c4) * x + c3) * x + c2) * x + c1) * x + c0
    return jnp.where(x < 3.0, p, 0.0)


def _bspline5_np(t):
    x = np.abs(t)
    x2 = x * x
    x4 = x2 * x2
    w1 = 11.0 / 20.0 - x2 / 2.0 + x4 / 4.0 - x4 * x / 12.0
    w2 = (17.0 / 40.0 + 5.0 * x / 8.0 - 7.0 * x2 / 4.0 + 5.0 * x2 * x / 4.0
          - 3.0 * x4 / 8.0 + x4 * x / 24.0)
    w3 = (3.0 - x) ** 5 / 120.0
    return np.where(x < 1.0, w1, np.where(x < 2.0, w2,
                    np.where(x < 3.0, w3, 0.0)))


@functools.lru_cache(maxsize=None)
def _spatial_weights(n_pix, n_grid):
    c = np.linspace(-0.5 - 0.25 / n_grid, n_grid - 1 + 0.5 + 0.25 / n_grid,
                    n_pix, dtype=np.float64)
    base = np.floor(c).astype(np.int64)
    W = np.zeros((n_pix, n_grid), np.float64)
    for i in range(6):
        n = base - 2 + i
        w = _bspline5_np(c - n)
        m = np.remainder(n, 2 * n_grid)
        refl = np.where(m >= n_grid, 2 * n_grid - 1 - m, m)
        np.add.at(W, (np.arange(n_pix), refl), w)
    return W.astype(np.float32)


def _hist_cdf_body(x_ref, cdf_ref, *, vox, n_bins, limit):
    tile = x_ref[0]
    bins = lax.broadcasted_iota(jnp.int32, (1, 1, n_bins), 2).astype(
        jnp.float32) * (1.0 / (n_bins - 1.0))
    z = (tile[:, :, None] - bins) * (1.0 / _BANDWIDTH)
    w = jnp.exp(-0.5 * (z * z))
    pdf = jnp.sum(w.reshape(vox, n_bins), axis=0, keepdims=True) / vox
    pdf = pdf / (jnp.sum(pdf) + 1e-10)
    histos = jnp.minimum(pdf * vox, limit)
    clipped = vox - jnp.sum(histos)
    residual = jnp.remainder(clipped, float(n_bins))
    redist = (clipped - residual) / n_bins
    bidx = lax.broadcasted_iota(jnp.int32, (1, n_bins), 1).astype(jnp.float32)
    histos = histos + redist + (bidx < residual).astype(jnp.float32)
    ii = lax.broadcasted_iota(jnp.int32, (n_bins, n_bins), 0)
    jj = lax.broadcasted_iota(jnp.int32, (n_bins, n_bins), 1)
    tri = (ii <= jj).astype(jnp.float32)
    cdf = jnp.dot(histos, tri, preferred_element_type=jnp.float32)
    cdf_ref[0] = jnp.clip(cdf * ((n_bins - 1.0) / vox), 0.0, n_bins - 1.0)


def _resample_body(x_ref, vol_ref, volc_ref, roww_ref, colw_ref, out_ref, *,
                   rows, w_pix, n_bins, n_tiles):
    f = x_ref[0] * (n_bins - 1.0)
    nodes = lax.broadcasted_iota(jnp.int32, (1, 1, n_bins), 2).astype(
        jnp.float32)
    wv = _bspline5_horner(f[:, :, None] - nodes)
    wv2 = wv.reshape(rows * w_pix, n_bins)
    vol = vol_ref[0]
    c = lax.dot_general(wv2, vol, (((1,), (1,)), ((), ())),
                        preferred_element_type=jnp.float32)
    ce = jnp.stack([_bspline5(f + 1.0), _bspline5(f + 2.0),
                    _bspline5(f - float(n_bins)),
                    _bspline5(f - float(n_bins + 1))],
                   axis=-1)
    ce2 = ce.reshape(rows * w_pix, 4)
    volc = volc_ref[0]
    c = c + lax.dot_general(ce2, volc, (((1,), (1,)), ((), ())),
                            preferred_element_type=jnp.float32)
    c3 = c.reshape(rows, w_pix, n_tiles)
    mask = roww_ref[...][:, None, :] * colw_ref[...][None, :, :]
    out_ref[0] = jnp.sum(c3 * mask, axis=2)


def _finalize_body(x_ref, o_ref):
    x = x_ref[...]
    mn = jnp.min(x)
    mx = jnp.max(x)
    o_ref[...] = (x - mn) / (mx - mn + 1e-10)


def kernel(x):
    B, C, H, W = x.shape
    th, tw = H // _GH, W // _GW
    vox = th * tw
    n_tiles = _GH * _GW
    bc = B * C
    nbt = bc * n_tiles
    limit = max(_CLIP_LIMIT * vox // _N_BINS, 1)

    xt = x.reshape(bc, _GH, th, _GW, tw).transpose(0, 1, 3, 2, 4)
    xt = xt.reshape(nbt, 8, vox // 8)
    cdfs = pl.pallas_call(
        functools.partial(_hist_cdf_body, vox=vox, n_bins=_N_BINS,
                          limit=float(limit)),
        grid=(nbt,),
        in_specs=[pl.BlockSpec((1, 8, vox // 8), lambda i: (i, 0, 0))],
        out_specs=pl.BlockSpec((1, 1, _N_BINS), lambda i: (i, 0, 0)),
        out_shape=jax.ShapeDtypeStruct((nbt, 1, _N_BINS), jnp.float32),
        compiler_params=pltpu.CompilerParams(
            dimension_semantics=("parallel",)),
    )(xt)

    vol = cdfs.reshape(bc, n_tiles, _N_BINS)
    vol_c = jnp.concatenate(
        [vol[:, :, 0:1], vol[:, :, 1:2],
         vol[:, :, _N_BINS - 1:_N_BINS], vol[:, :, _N_BINS - 2:_N_BINS - 1]],
        axis=-1)

    roww = np.repeat(_spatial_weights(H, _GH), _GW, axis=1)
    colw = np.tile(_spatial_weights(W, _GW), (1, _GH))

    ROWS = 8
    n_rb = H // ROWS
    out = pl.pallas_call(
        functools.partial(_resample_body, rows=ROWS, w_pix=W,
                          n_bins=_N_BINS, n_tiles=n_tiles),
        grid=(bc, n_rb),
        in_specs=[
            pl.BlockSpec((1, ROWS, W), lambda b, r: (b, r, 0)),
            pl.BlockSpec((1, n_tiles, _N_BINS), lambda b, r: (b, 0, 0)),
            pl.BlockSpec((1, n_tiles, 4), lambda b, r: (b, 0, 0)),
            pl.BlockSpec((ROWS, n_tiles), lambda b, r: (r, 0)),
            pl.BlockSpec((W, n_tiles), lambda b, r: (0, 0)),
        ],
        out_specs=pl.BlockSpec((1, ROWS, W), lambda b, r: (b, r, 0)),
        out_shape=jax.ShapeDtypeStruct((bc, H, W), jnp.float32),
        compiler_params=pltpu.CompilerParams(
            dimension_semantics=("parallel", "parallel")),
    )(x.reshape(bc, H, W), vol, vol_c, jnp.asarray(roww), jnp.asarray(colw))

    outn = pl.pallas_call(
        _finalize_body,
        out_shape=jax.ShapeDtypeStruct((bc, H, W), jnp.float32),
    )(out)
    return outn.reshape(B, C, H, W)
